# baseline (speedup 1.0000x reference)
import functools

import jax
import jax.numpy as jnp
from jax.experimental import pallas as pl

_N = 16384
_K = 1024
_DIM = 256
_BN = 512


def _cdist_mean_kernel(x_ref, c_ref, out_ref):
    i = pl.program_id(0)

    @pl.when(i == 0)
    def _init():
        out_ref[...] = jnp.zeros((1, 1), jnp.float32)

    xb = x_ref[...]
    cb = c_ref[...]
    x2 = jnp.sum(xb * xb, axis=1, keepdims=True)
    c2 = jnp.sum(cb * cb, axis=1)[None, :]
    dot = jax.lax.dot_general(
        xb.astype(jnp.bfloat16),
        cb.astype(jnp.bfloat16),
        dimension_numbers=(((1,), (1,)), ((), ())),
        preferred_element_type=jnp.float32,
    )
    d2 = (x2 + c2) - 2.0 * dot
    dist = jnp.sqrt(jnp.maximum(d2, 1e-12))
    out_ref[...] += (jnp.sum(dist) * jnp.float32(1.0 / (_N * _K)))[None, None]


@jax.jit
def kernel(x, codebook):
    out = pl.pallas_call(
        _cdist_mean_kernel,
        grid=(_N // _BN,),
        in_specs=[
            pl.BlockSpec((_BN, _DIM), lambda i: (i, 0)),
            pl.BlockSpec((_K, _DIM), lambda i: (0, 0)),
        ],
        out_specs=pl.BlockSpec((1, 1), lambda i: (0, 0)),
        out_shape=jax.ShapeDtypeStruct((1, 1), jnp.float32),
    )(x, codebook)
    return out[0, 0]


# speedup vs baseline: 1.4702x; 1.4702x over previous
import jax
import jax.numpy as jnp
from jax.experimental import pallas as pl
from jax.experimental.pallas import tpu as pltpu

_N = 16384
_K = 1024
_DIM = 256
_BN = 512
_STEPS = _N // _BN


def _cdist_mean_kernel(x_ref, c_ref, out_ref, cb_ref, c2_ref, acc_ref):
    i = pl.program_id(0)

    @pl.when(i == 0)
    def _init():
        cf = c_ref[...]
        cb_ref[...] = cf.astype(jnp.bfloat16)
        c2_ref[...] = jnp.sum(cf * cf, axis=1)[None, :]
        acc_ref[...] = jnp.zeros((8, _K), jnp.float32)

    xb = x_ref[...]
    x2 = jnp.sum(xb * xb, axis=1, keepdims=True)
    dot = jax.lax.dot_general(
        (xb * -2.0).astype(jnp.bfloat16),
        cb_ref[...],
        dimension_numbers=(((1,), (1,)), ((), ())),
        preferred_element_type=jnp.float32,
    )
    d2 = jnp.maximum(dot + (x2 + c2_ref[...]), 1e-12)
    dist = d2 * jax.lax.rsqrt(d2)
    acc_ref[...] += jnp.sum(dist.reshape(_BN // 8, 8, _K), axis=0)

    @pl.when(i == _STEPS - 1)
    def _final():
        out_ref[...] = (jnp.sum(acc_ref[...]) * jnp.float32(1.0 / (_N * _K)))[
            None, None
        ]


@jax.jit
def kernel(x, codebook):
    out = pl.pallas_call(
        _cdist_mean_kernel,
        grid=(_STEPS,),
        in_specs=[
            pl.BlockSpec((_BN, _DIM), lambda i: (i, 0)),
            pl.BlockSpec((_K, _DIM), lambda i: (0, 0)),
        ],
        out_specs=pl.BlockSpec((1, 1), lambda i: (0, 0)),
        out_shape=jax.ShapeDtypeStruct((1, 1), jnp.float32),
        scratch_shapes=[
            pltpu.VMEM((_K, _DIM), jnp.bfloat16),
            pltpu.VMEM((1, _K), jnp.float32),
            pltpu.VMEM((8, _K), jnp.float32),
        ],
    )(x, codebook)
    return out[0, 0]
